# baseline (speedup 1.0000x reference)
import functools

import jax
import jax.numpy as jnp
from jax import lax
from jax.experimental import pallas as pl
from jax.experimental.pallas import tpu as pltpu
from jax.experimental.pallas import tpu_sc as plsc

L = 16
D = 128
H = 128
BLK = 512

NC = 2
NS = 16
NW = NC * NS
CHUNK = 128
N_ACC = 10240


def _segsum_body(y_hbm, src_hbm, dst_hbm, z_hbm, out_hbm,
                 src_v, dst_v, rows_v, acc_sh, sem):
    c = lax.axis_index("c")
    s = lax.axis_index("s")
    wid = c * NS + s
    rows_per_tile = N_ACC // NS
    nchunks = src_hbm.shape[0] // (NW * CHUNK)
    pltpu.sync_copy(z_hbm, acc_sh.at[pl.ds(s * rows_per_tile, rows_per_tile)])
    plsc.subcore_barrier()
    base0 = wid * nchunks * CHUNK

    def chunk(i, carry):
        base = pl.multiple_of(base0 + i * CHUNK, CHUNK)
        pltpu.sync_copy(src_hbm.at[pl.ds(base, CHUNK)], src_v)
        pltpu.async_copy(y_hbm.at[src_v], rows_v, sem).wait()
        pltpu.sync_copy(dst_hbm.at[pl.ds(base, CHUNK)], dst_v)
        pltpu.sync_copy(rows_v, acc_sh.at[dst_v], add=True)
        return carry

    lax.fori_loop(0, nchunks, chunk, 0)
    plsc.subcore_barrier()
    pltpu.sync_copy(acc_sh.at[pl.ds(s * rows_per_tile, rows_per_tile)],
                    out_hbm.at[c, pl.ds(s * rows_per_tile, rows_per_tile)])


def _segsum_sc(y, src_p, dst_p):
    mesh = plsc.VectorSubcoreMesh(core_axis_name="c", subcore_axis_name="s")
    z = jnp.zeros((N_ACC // NS, D), jnp.float32)
    f = functools.partial(
        pl.kernel, mesh=mesh,
        out_type=jax.ShapeDtypeStruct((NC, N_ACC, D), jnp.float32),
        name="segsum",
        scratch_types=[
            pltpu.VMEM((CHUNK,), jnp.int32),
            pltpu.VMEM((CHUNK,), jnp.int32),
            pltpu.VMEM((CHUNK, D), jnp.float32),
            pltpu.VMEM_SHARED((N_ACC, D), jnp.float32),
            pltpu.SemaphoreType.DMA,
        ],
    )(_segsum_body)
    return f(y, src_p, dst_p, z)


def _deg_body(dst_hbm, ones_hbm, z_hbm, out_hbm,
              ones_v, dst_v, acc_sh):
    c = lax.axis_index("c")
    s = lax.axis_index("s")
    wid = c * NS + s
    rows_per_tile = N_ACC // NS
    nchunks = dst_hbm.shape[0] // (NW * CHUNK)
    pltpu.sync_copy(z_hbm, acc_sh.at[pl.ds(s * rows_per_tile, rows_per_tile)])
    pltpu.sync_copy(ones_hbm, ones_v)
    plsc.subcore_barrier()
    base0 = wid * nchunks * CHUNK

    def chunk(i, carry):
        base = pl.multiple_of(base0 + i * CHUNK, CHUNK)
        pltpu.sync_copy(dst_hbm.at[pl.ds(base, CHUNK)], dst_v)
        pltpu.sync_copy(ones_v, acc_sh.at[dst_v], add=True)
        return carry

    lax.fori_loop(0, nchunks, chunk, 0)
    plsc.subcore_barrier()
    pltpu.sync_copy(acc_sh.at[pl.ds(s * rows_per_tile, rows_per_tile)],
                    out_hbm.at[c, pl.ds(s * rows_per_tile, rows_per_tile)])


def _deg_sc(dst_p):
    mesh = plsc.VectorSubcoreMesh(core_axis_name="c", subcore_axis_name="s")
    z = jnp.zeros((N_ACC // NS,), jnp.float32)
    ones = jnp.ones((CHUNK,), jnp.float32)
    f = functools.partial(
        pl.kernel, mesh=mesh,
        out_type=jax.ShapeDtypeStruct((NC, N_ACC), jnp.float32),
        scratch_types=[
            pltpu.VMEM((CHUNK,), jnp.float32),
            pltpu.VMEM((CHUNK,), jnp.int32),
            pltpu.VMEM_SHARED((N_ACC,), jnp.float32),
        ],
    )(_deg_body)
    return f(dst_p, ones, z)


def _gru_body(emb_ref, h0_ref, wih_ref, whh_ref, bih_ref, bhh_ref, out_ref,
              gi_ref):
    blk = h0_ref.shape[0]
    x_all = emb_ref[...].reshape(L * blk, D).astype(jnp.bfloat16)
    gi_all = jnp.dot(x_all, wih_ref[...].astype(jnp.bfloat16),
                     preferred_element_type=jnp.float32)
    gi_ref[...] = (gi_all + bih_ref[...]).reshape(L, blk, 3 * H)
    whh = whh_ref[...].astype(jnp.bfloat16)
    bhh = bhh_ref[...]

    def step(t, h):
        gi = gi_ref[t]
        gh = jnp.dot(h.astype(jnp.bfloat16), whh,
                     preferred_element_type=jnp.float32) + bhh
        r = jax.nn.sigmoid(gi[:, :H] + gh[:, :H])
        z = jax.nn.sigmoid(gi[:, H:2 * H] + gh[:, H:2 * H])
        n = jnp.tanh(gi[:, 2 * H:] + r * gh[:, 2 * H:])
        return n + z * (h - n)

    out_ref[...] = jax.lax.fori_loop(0, L, step, h0_ref[...])


def _gru(emb_seq, h0p, wih_t, whh_t, bih, bhh):
    n_pad = emb_seq.shape[1]
    return pl.pallas_call(
        _gru_body,
        grid=(n_pad // BLK,),
        in_specs=[
            pl.BlockSpec((L, BLK, D), lambda i: (0, i, 0)),
            pl.BlockSpec((BLK, H), lambda i: (i, 0)),
            pl.BlockSpec((D, 3 * H), lambda i: (0, 0)),
            pl.BlockSpec((H, 3 * H), lambda i: (0, 0)),
            pl.BlockSpec((1, 3 * H), lambda i: (0, 0)),
            pl.BlockSpec((1, 3 * H), lambda i: (0, 0)),
        ],
        out_specs=pl.BlockSpec((BLK, H), lambda i: (i, 0)),
        out_shape=jax.ShapeDtypeStruct((n_pad, H), jnp.float32),
        scratch_shapes=[pltpu.VMEM((L, BLK, 3 * H), jnp.float32)],
        compiler_params=pltpu.CompilerParams(
            dimension_semantics=("arbitrary",)),
    )(emb_seq, h0p, wih_t, whh_t, bih, bhh)


def kernel(user_text, user_feats, graph_node_features, graph_edge_index,
           merged_tree_feature, merged_tree_edge_index, indices,
           emb_table, h0, W_ih, W_hh, b_ih, b_hh,
           W1, b1, W2, b2, Wf, bf):
    n = merged_tree_feature.shape[0]
    b_trees = user_text.shape[0]
    n_pad = ((n + BLK - 1) // BLK) * BLK
    pad = n_pad - n

    toks = jnp.pad(merged_tree_feature.T, ((0, 0), (0, pad)))
    emb_seq = jnp.take(emb_table, toks.reshape(-1), axis=0).reshape(
        L, n_pad, D)
    h0p = jnp.pad(h0, ((0, pad), (0, 0)))
    x1 = _gru(emb_seq, h0p, W_ih.T, W_hh.T, b_ih[None, :], b_hh[None, :])[:n]

    src = merged_tree_edge_index[0].astype(jnp.int32)
    dst = merged_tree_edge_index[1].astype(jnp.int32)
    e = src.shape[0]
    quant = NW * CHUNK
    e_pad = ((e + quant - 1) // quant) * quant
    src_p = jnp.concatenate(
        [src, jnp.zeros((e_pad - e,), jnp.int32)])
    dst_p = jnp.concatenate(
        [dst, jnp.full((e_pad - e,), N_ACC - 1, jnp.int32)])

    degp = _deg_sc(dst_p)
    deg = degp[0, :n] + degp[1, :n] + 1.0
    dinv = jax.lax.rsqrt(deg)[:, None]

    xw1 = x1 @ W1
    y1 = xw1 * dinv
    s1p = _segsum_sc(y1, src_p, dst_p)
    s1 = s1p[0, :n] + s1p[1, :n]
    x2 = dinv * s1 + dinv * dinv * xw1 + b1

    xcat = jax.nn.relu(
        jnp.concatenate([x2, jnp.take(x1, indices, axis=0)], axis=1))
    xw2 = xcat @ W2
    y2 = xw2 * dinv
    s2p = _segsum_sc(y2, src_p, dst_p)
    s2 = s2p[0, :n] + s2p[1, :n]
    x3 = jax.nn.relu(dinv * s2 + dinv * dinv * xw2 + b2)

    xf = jnp.concatenate([x3, jnp.take(x2, indices, axis=0)], axis=1)
    sums = jax.ops.segment_sum(xf, indices, num_segments=b_trees)
    cnt = jax.ops.segment_sum(jnp.ones((n,), xf.dtype), indices,
                              num_segments=b_trees)
    mean = sums / jnp.clip(cnt, 1.0, None)[:, None]
    return mean @ Wf + bf


# speedup vs baseline: 1.0036x; 1.0036x over previous
import functools

import jax
import jax.numpy as jnp
from jax import lax
from jax.experimental import pallas as pl
from jax.experimental.pallas import tpu as pltpu
from jax.experimental.pallas import tpu_sc as plsc

L = 16
D = 128
H = 128
BLK = 512

NC = 2
NS = 16
NW = NC * NS
CHUNK = 128
N_ACC = 10240


def _segsum_body(y_hbm, src_hbm, dst_hbm, z_hbm, out_hbm,
                 src_v, dst_v, rows_v, acc_sh, sem):
    c = lax.axis_index("c")
    s = lax.axis_index("s")
    wid = c * NS + s
    rows_per_tile = N_ACC // NS
    nchunks = src_hbm.shape[0] // (NW * CHUNK)
    pltpu.sync_copy(z_hbm, acc_sh.at[pl.ds(s * rows_per_tile, rows_per_tile)])
    plsc.subcore_barrier()
    base0 = wid * nchunks * CHUNK

    def chunk(i, carry):
        base = pl.multiple_of(base0 + i * CHUNK, CHUNK)
        pltpu.sync_copy(src_hbm.at[pl.ds(base, CHUNK)], src_v)
        pltpu.async_copy(y_hbm.at[src_v], rows_v, sem).wait()
        pltpu.sync_copy(dst_hbm.at[pl.ds(base, CHUNK)], dst_v)
        pltpu.sync_copy(rows_v, acc_sh.at[dst_v], add=True)
        return carry

    lax.fori_loop(0, nchunks, chunk, 0)
    plsc.subcore_barrier()
    pltpu.sync_copy(acc_sh.at[pl.ds(s * rows_per_tile, rows_per_tile)],
                    out_hbm.at[c, pl.ds(s * rows_per_tile, rows_per_tile)])


def _segsum_sc(y, src_p, dst_p):
    mesh = plsc.VectorSubcoreMesh(core_axis_name="c", subcore_axis_name="s")
    z = jnp.zeros((N_ACC // NS, D), jnp.float32)
    f = functools.partial(
        pl.kernel, mesh=mesh,
        out_type=jax.ShapeDtypeStruct((NC, N_ACC, D), jnp.float32),
        name="segsum",
        scratch_types=[
            pltpu.VMEM((CHUNK,), jnp.int32),
            pltpu.VMEM((CHUNK,), jnp.int32),
            pltpu.VMEM((CHUNK, D), jnp.float32),
            pltpu.VMEM_SHARED((N_ACC, D), jnp.float32),
            pltpu.SemaphoreType.DMA,
        ],
    )(_segsum_body)
    return f(y, src_p, dst_p, z)


def _deg_body(dst_hbm, ones_hbm, z_hbm, out_hbm,
              ones_v, dst_v, acc_sh):
    c = lax.axis_index("c")
    s = lax.axis_index("s")
    wid = c * NS + s
    rows_per_tile = N_ACC // NS
    nchunks = dst_hbm.shape[0] // (NW * CHUNK)
    pltpu.sync_copy(z_hbm, acc_sh.at[pl.ds(s * rows_per_tile, rows_per_tile)])
    pltpu.sync_copy(ones_hbm, ones_v)
    plsc.subcore_barrier()
    base0 = wid * nchunks * CHUNK

    def chunk(i, carry):
        base = pl.multiple_of(base0 + i * CHUNK, CHUNK)
        pltpu.sync_copy(dst_hbm.at[pl.ds(base, CHUNK)], dst_v)
        pltpu.sync_copy(ones_v, acc_sh.at[dst_v], add=True)
        return carry

    lax.fori_loop(0, nchunks, chunk, 0)
    plsc.subcore_barrier()
    pltpu.sync_copy(acc_sh.at[pl.ds(s * rows_per_tile, rows_per_tile)],
                    out_hbm.at[c, pl.ds(s * rows_per_tile, rows_per_tile)])


def _deg_sc(dst_p):
    mesh = plsc.VectorSubcoreMesh(core_axis_name="c", subcore_axis_name="s")
    z = jnp.zeros((N_ACC // NS,), jnp.float32)
    ones = jnp.ones((CHUNK,), jnp.float32)
    f = functools.partial(
        pl.kernel, mesh=mesh,
        out_type=jax.ShapeDtypeStruct((NC, N_ACC), jnp.float32),
        scratch_types=[
            pltpu.VMEM((CHUNK,), jnp.float32),
            pltpu.VMEM((CHUNK,), jnp.int32),
            pltpu.VMEM_SHARED((N_ACC,), jnp.float32),
        ],
    )(_deg_body)
    return f(dst_p, ones, z)


def _gru_body(emb_ref, h0_ref, wih_ref, whh_ref, bih_ref, bhh_ref, out_ref,
              gi_ref):
    blk = h0_ref.shape[0]
    x_all = emb_ref[...].reshape(L * blk, D)
    gi_all = jnp.dot(x_all, wih_ref[...],
                     preferred_element_type=jnp.float32)
    gi_ref[...] = (gi_all + bih_ref[...]).reshape(L, blk, 3 * H)
    whh = whh_ref[...]
    bhh = bhh_ref[...]

    def step(t, h):
        gi = gi_ref[t]
        gh = jnp.dot(h, whh, preferred_element_type=jnp.float32) + bhh
        r = jax.nn.sigmoid(gi[:, :H] + gh[:, :H])
        z = jax.nn.sigmoid(gi[:, H:2 * H] + gh[:, H:2 * H])
        n = jnp.tanh(gi[:, 2 * H:] + r * gh[:, 2 * H:])
        return n + z * (h - n)

    out_ref[...] = jax.lax.fori_loop(0, L, step, h0_ref[...])


def _gru(emb_seq, h0p, wih_t, whh_t, bih, bhh):
    n_pad = emb_seq.shape[1]
    return pl.pallas_call(
        _gru_body,
        grid=(n_pad // BLK,),
        in_specs=[
            pl.BlockSpec((L, BLK, D), lambda i: (0, i, 0)),
            pl.BlockSpec((BLK, H), lambda i: (i, 0)),
            pl.BlockSpec((D, 3 * H), lambda i: (0, 0)),
            pl.BlockSpec((H, 3 * H), lambda i: (0, 0)),
            pl.BlockSpec((1, 3 * H), lambda i: (0, 0)),
            pl.BlockSpec((1, 3 * H), lambda i: (0, 0)),
        ],
        out_specs=pl.BlockSpec((BLK, H), lambda i: (i, 0)),
        out_shape=jax.ShapeDtypeStruct((n_pad, H), jnp.float32),
        scratch_shapes=[pltpu.VMEM((L, BLK, 3 * H), jnp.float32)],
        compiler_params=pltpu.CompilerParams(
            dimension_semantics=("arbitrary",)),
    )(emb_seq, h0p, wih_t, whh_t, bih, bhh)


def kernel(user_text, user_feats, graph_node_features, graph_edge_index,
           merged_tree_feature, merged_tree_edge_index, indices,
           emb_table, h0, W_ih, W_hh, b_ih, b_hh,
           W1, b1, W2, b2, Wf, bf):
    n = merged_tree_feature.shape[0]
    b_trees = user_text.shape[0]
    n_pad = ((n + BLK - 1) // BLK) * BLK
    pad = n_pad - n

    toks = jnp.pad(merged_tree_feature.T, ((0, 0), (0, pad)))
    emb_seq = jnp.take(emb_table, toks.reshape(-1), axis=0).reshape(
        L, n_pad, D)
    h0p = jnp.pad(h0, ((0, pad), (0, 0)))
    x1 = _gru(emb_seq, h0p, W_ih.T, W_hh.T, b_ih[None, :], b_hh[None, :])[:n]

    src = merged_tree_edge_index[0].astype(jnp.int32)
    dst = merged_tree_edge_index[1].astype(jnp.int32)
    e = src.shape[0]
    quant = NW * CHUNK
    e_pad = ((e + quant - 1) // quant) * quant
    src_p = jnp.concatenate(
        [src, jnp.zeros((e_pad - e,), jnp.int32)])
    dst_p = jnp.concatenate(
        [dst, jnp.full((e_pad - e,), N_ACC - 1, jnp.int32)])

    degp = _deg_sc(dst_p)
    deg = degp[0, :n] + degp[1, :n] + 1.0
    dinv = jax.lax.rsqrt(deg)[:, None]

    xw1 = x1 @ W1
    y1 = xw1 * dinv
    s1p = _segsum_sc(y1, src_p, dst_p)
    s1 = s1p[0, :n] + s1p[1, :n]
    x2 = dinv * s1 + dinv * dinv * xw1 + b1

    xcat = jax.nn.relu(
        jnp.concatenate([x2, jnp.take(x1, indices, axis=0)], axis=1))
    xw2 = xcat @ W2
    y2 = xw2 * dinv
    s2p = _segsum_sc(y2, src_p, dst_p)
    s2 = s2p[0, :n] + s2p[1, :n]
    x3 = jax.nn.relu(dinv * s2 + dinv * dinv * xw2 + b2)

    xf = jnp.concatenate([x3, jnp.take(x2, indices, axis=0)], axis=1)
    sums = jax.ops.segment_sum(xf, indices, num_segments=b_trees)
    cnt = jax.ops.segment_sum(jnp.ones((n,), xf.dtype), indices,
                              num_segments=b_trees)
    mean = sums / jnp.clip(cnt, 1.0, None)[:, None]
    return mean @ Wf + bf


# speedup vs baseline: 1.0950x; 1.0910x over previous
import functools

import jax
import jax.numpy as jnp
from jax import lax
from jax.experimental import pallas as pl
from jax.experimental.pallas import tpu as pltpu
from jax.experimental.pallas import tpu_sc as plsc

L = 16
D = 128
H = 128
BLK = 512

NC = 2
NS = 16
NW = NC * NS
CHUNK = 128
N_ACC = 10240


def _segsum_body(y_hbm, src_hbm, dst_hbm, z_hbm, out_hbm,
                 src_v, dst_v, rows_v, acc_sh, sem):
    c = lax.axis_index("c")
    s = lax.axis_index("s")
    wid = c * NS + s
    rows_per_tile = N_ACC // NS
    nchunks = src_hbm.shape[0] // (NW * CHUNK)
    pltpu.sync_copy(z_hbm, acc_sh.at[pl.ds(s * rows_per_tile, rows_per_tile)])
    plsc.subcore_barrier()
    base0 = wid * nchunks * CHUNK

    def chunk(i, carry):
        base = pl.multiple_of(base0 + i * CHUNK, CHUNK)
        pltpu.sync_copy(src_hbm.at[pl.ds(base, CHUNK)], src_v)
        pltpu.async_copy(y_hbm.at[src_v], rows_v, sem).wait()
        pltpu.sync_copy(dst_hbm.at[pl.ds(base, CHUNK)], dst_v)
        pltpu.sync_copy(rows_v, acc_sh.at[dst_v], add=True)
        return carry

    lax.fori_loop(0, nchunks, chunk, 0)
    plsc.subcore_barrier()
    pltpu.sync_copy(acc_sh.at[pl.ds(s * rows_per_tile, rows_per_tile)],
                    out_hbm.at[c, pl.ds(s * rows_per_tile, rows_per_tile)])


def _segsum_sc(y, src_p, dst_p):
    mesh = plsc.VectorSubcoreMesh(core_axis_name="c", subcore_axis_name="s")
    z = jnp.zeros((N_ACC // NS, D), jnp.float32)
    f = functools.partial(
        pl.kernel, mesh=mesh,
        out_type=jax.ShapeDtypeStruct((NC, N_ACC, D), jnp.float32),
        name="segsum",
        scratch_types=[
            pltpu.VMEM((CHUNK,), jnp.int32),
            pltpu.VMEM((CHUNK,), jnp.int32),
            pltpu.VMEM((CHUNK, D), jnp.float32),
            pltpu.VMEM_SHARED((N_ACC, D), jnp.float32),
            pltpu.SemaphoreType.DMA,
        ],
    )(_segsum_body)
    return f(y, src_p, dst_p, z)


def _deg_body(dst_hbm, ones_hbm, z_hbm, out_hbm,
              ones_v, dst_v, acc_sh):
    c = lax.axis_index("c")
    s = lax.axis_index("s")
    wid = c * NS + s
    rows_per_tile = N_ACC // NS
    nchunks = dst_hbm.shape[0] // (NW * CHUNK)
    pltpu.sync_copy(z_hbm, acc_sh.at[pl.ds(s * rows_per_tile, rows_per_tile)])
    pltpu.sync_copy(ones_hbm, ones_v)
    plsc.subcore_barrier()
    base0 = wid * nchunks * CHUNK

    def chunk(i, carry):
        base = pl.multiple_of(base0 + i * CHUNK, CHUNK)
        pltpu.sync_copy(dst_hbm.at[pl.ds(base, CHUNK)], dst_v)
        pltpu.sync_copy(ones_v, acc_sh.at[dst_v], add=True)
        return carry

    lax.fori_loop(0, nchunks, chunk, 0)
    plsc.subcore_barrier()
    pltpu.sync_copy(acc_sh.at[pl.ds(s * rows_per_tile, rows_per_tile)],
                    out_hbm.at[c, pl.ds(s * rows_per_tile, rows_per_tile)])


def _deg_sc(dst_p):
    mesh = plsc.VectorSubcoreMesh(core_axis_name="c", subcore_axis_name="s")
    z = jnp.zeros((N_ACC // NS,), jnp.float32)
    ones = jnp.ones((CHUNK,), jnp.float32)
    f = functools.partial(
        pl.kernel, mesh=mesh,
        out_type=jax.ShapeDtypeStruct((NC, N_ACC), jnp.float32),
        scratch_types=[
            pltpu.VMEM((CHUNK,), jnp.float32),
            pltpu.VMEM((CHUNK,), jnp.int32),
            pltpu.VMEM_SHARED((N_ACC,), jnp.float32),
        ],
    )(_deg_body)
    return f(dst_p, ones, z)


def _gru_body(emb_ref, h0_ref, wih_ref, whh_ref, bih_ref, bhh_ref, out_ref,
              gi_ref):
    blk = h0_ref.shape[0]
    x_all = emb_ref[...].reshape(L * blk, D)
    gi_all = jnp.dot(x_all, wih_ref[...],
                     preferred_element_type=jnp.float32)
    gi_ref[...] = (gi_all + bih_ref[...]).reshape(L, blk, 3 * H)
    whh = whh_ref[...]
    bhh = bhh_ref[...]

    def step(t, h):
        gi = gi_ref[t]
        gh = jnp.dot(h, whh, preferred_element_type=jnp.float32) + bhh
        r = jax.nn.sigmoid(gi[:, :H] + gh[:, :H])
        z = jax.nn.sigmoid(gi[:, H:2 * H] + gh[:, H:2 * H])
        n = jnp.tanh(gi[:, 2 * H:] + r * gh[:, 2 * H:])
        return n + z * (h - n)

    out_ref[...] = jax.lax.fori_loop(0, L, step, h0_ref[...])


def _gru(emb_seq, h0p, wih_t, whh_t, bih, bhh):
    n_pad = emb_seq.shape[1]
    return pl.pallas_call(
        _gru_body,
        grid=(n_pad // BLK,),
        in_specs=[
            pl.BlockSpec((L, BLK, D), lambda i: (0, i, 0)),
            pl.BlockSpec((BLK, H), lambda i: (i, 0)),
            pl.BlockSpec((D, 3 * H), lambda i: (0, 0)),
            pl.BlockSpec((H, 3 * H), lambda i: (0, 0)),
            pl.BlockSpec((1, 3 * H), lambda i: (0, 0)),
            pl.BlockSpec((1, 3 * H), lambda i: (0, 0)),
        ],
        out_specs=pl.BlockSpec((BLK, H), lambda i: (i, 0)),
        out_shape=jax.ShapeDtypeStruct((n_pad, H), jnp.float32),
        scratch_shapes=[pltpu.VMEM((L, BLK, 3 * H), jnp.float32)],
        compiler_params=pltpu.CompilerParams(
            dimension_semantics=("arbitrary",)),
    )(emb_seq, h0p, wih_t, whh_t, bih, bhh)


def kernel(user_text, user_feats, graph_node_features, graph_edge_index,
           merged_tree_feature, merged_tree_edge_index, indices,
           emb_table, h0, W_ih, W_hh, b_ih, b_hh,
           W1, b1, W2, b2, Wf, bf):
    n = merged_tree_feature.shape[0]
    b_trees = user_text.shape[0]
    n_pad = ((n + BLK - 1) // BLK) * BLK
    pad = n_pad - n

    emb = jnp.take(emb_table, merged_tree_feature.reshape(-1), axis=0)
    emb_seq = jnp.transpose(emb.reshape(n, L, D), (1, 0, 2))
    emb_seq = jnp.pad(emb_seq, ((0, 0), (0, pad), (0, 0)))
    h0p = jnp.pad(h0, ((0, pad), (0, 0)))
    x1 = _gru(emb_seq, h0p, W_ih.T, W_hh.T, b_ih[None, :], b_hh[None, :])[:n]

    src = merged_tree_edge_index[0].astype(jnp.int32)
    dst = merged_tree_edge_index[1].astype(jnp.int32)
    e = src.shape[0]
    quant = NW * CHUNK
    e_pad = ((e + quant - 1) // quant) * quant
    src_p = jnp.concatenate(
        [src, jnp.zeros((e_pad - e,), jnp.int32)])
    dst_p = jnp.concatenate(
        [dst, jnp.full((e_pad - e,), N_ACC - 1, jnp.int32)])

    degp = _deg_sc(dst_p)
    deg = degp[0, :n] + degp[1, :n] + 1.0
    dinv = jax.lax.rsqrt(deg)[:, None]

    xw1 = x1 @ W1
    y1 = xw1 * dinv
    s1p = _segsum_sc(y1, src_p, dst_p)
    s1 = s1p[0, :n] + s1p[1, :n]
    x2 = dinv * s1 + dinv * dinv * xw1 + b1

    xcat = jax.nn.relu(
        jnp.concatenate([x2, jnp.take(x1, indices, axis=0)], axis=1))
    xw2 = xcat @ W2
    y2 = xw2 * dinv
    s2p = _segsum_sc(y2, src_p, dst_p)
    s2 = s2p[0, :n] + s2p[1, :n]
    x3 = jax.nn.relu(dinv * s2 + dinv * dinv * xw2 + b2)

    xf = jnp.concatenate([x3, jnp.take(x2, indices, axis=0)], axis=1)
    sums = jax.ops.segment_sum(xf, indices, num_segments=b_trees)
    cnt = jax.ops.segment_sum(jnp.ones((n,), xf.dtype), indices,
                              num_segments=b_trees)
    mean = sums / jnp.clip(cnt, 1.0, None)[:, None]
    return mean @ Wf + bf


# speedup vs baseline: 1.1912x; 1.0879x over previous
import functools

import jax
import jax.numpy as jnp
from jax import lax
from jax.experimental import pallas as pl
from jax.experimental.pallas import tpu as pltpu
from jax.experimental.pallas import tpu_sc as plsc

L = 16
D = 128
H = 128
BLK = 512

NC = 2
NS = 16
NW = NC * NS
CHUNK = 128
N_ACC = 10240


def _segsum_body(y_hbm, src_hbm, dst_hbm, z_hbm, out_hbm,
                 src_v, dst_v, rows_v, acc_sh, sem):
    c = lax.axis_index("c")
    s = lax.axis_index("s")
    wid = c * NS + s
    rows_per_tile = N_ACC // NS
    nchunks = src_hbm.shape[0] // (NW * CHUNK)
    pltpu.sync_copy(z_hbm, acc_sh.at[pl.ds(s * rows_per_tile, rows_per_tile)])
    plsc.subcore_barrier()
    base0 = wid * nchunks * CHUNK

    def chunk(i, carry):
        base = pl.multiple_of(base0 + i * CHUNK, CHUNK)
        pltpu.sync_copy(src_hbm.at[pl.ds(base, CHUNK)], src_v)
        pltpu.async_copy(y_hbm.at[src_v], rows_v, sem).wait()
        pltpu.sync_copy(dst_hbm.at[pl.ds(base, CHUNK)], dst_v)
        pltpu.sync_copy(rows_v, acc_sh.at[dst_v], add=True)
        return carry

    lax.fori_loop(0, nchunks, chunk, 0)
    plsc.subcore_barrier()
    pltpu.sync_copy(acc_sh.at[pl.ds(s * rows_per_tile, rows_per_tile)],
                    out_hbm.at[c, pl.ds(s * rows_per_tile, rows_per_tile)])


def _segsum_sc(y, src_p, dst_p):
    mesh = plsc.VectorSubcoreMesh(core_axis_name="c", subcore_axis_name="s")
    z = jnp.zeros((N_ACC // NS, D), jnp.float32)
    f = functools.partial(
        pl.kernel, mesh=mesh,
        out_type=jax.ShapeDtypeStruct((NC, N_ACC, D), jnp.float32),
        name="segsum",
        scratch_types=[
            pltpu.VMEM((CHUNK,), jnp.int32),
            pltpu.VMEM((CHUNK,), jnp.int32),
            pltpu.VMEM((CHUNK, D), jnp.float32),
            pltpu.VMEM_SHARED((N_ACC, D), jnp.float32),
            pltpu.SemaphoreType.DMA,
        ],
    )(_segsum_body)
    return f(y, src_p, dst_p, z)


def _deg_body(dst_hbm, ones_hbm, z_hbm, out_hbm,
              ones_v, dst_v, acc_sh):
    c = lax.axis_index("c")
    s = lax.axis_index("s")
    wid = c * NS + s
    rows_per_tile = N_ACC // NS
    nchunks = dst_hbm.shape[0] // (NW * CHUNK)
    pltpu.sync_copy(z_hbm, acc_sh.at[pl.ds(s * rows_per_tile, rows_per_tile)])
    pltpu.sync_copy(ones_hbm, ones_v)
    plsc.subcore_barrier()
    base0 = wid * nchunks * CHUNK

    def chunk(i, carry):
        base = pl.multiple_of(base0 + i * CHUNK, CHUNK)
        pltpu.sync_copy(dst_hbm.at[pl.ds(base, CHUNK)], dst_v)
        pltpu.sync_copy(ones_v, acc_sh.at[dst_v], add=True)
        return carry

    lax.fori_loop(0, nchunks, chunk, 0)
    plsc.subcore_barrier()
    pltpu.sync_copy(acc_sh.at[pl.ds(s * rows_per_tile, rows_per_tile)],
                    out_hbm.at[c, pl.ds(s * rows_per_tile, rows_per_tile)])


def _deg_sc(dst_p):
    mesh = plsc.VectorSubcoreMesh(core_axis_name="c", subcore_axis_name="s")
    z = jnp.zeros((N_ACC // NS,), jnp.float32)
    ones = jnp.ones((CHUNK,), jnp.float32)
    f = functools.partial(
        pl.kernel, mesh=mesh,
        out_type=jax.ShapeDtypeStruct((NC, N_ACC), jnp.float32),
        scratch_types=[
            pltpu.VMEM((CHUNK,), jnp.float32),
            pltpu.VMEM((CHUNK,), jnp.int32),
            pltpu.VMEM_SHARED((N_ACC,), jnp.float32),
        ],
    )(_deg_body)
    return f(dst_p, ones, z)


def _gru_body(emb_ref, h0_ref, wih_ref, whh_ref, bih_ref, bhh_ref, out_ref,
              gi_ref):
    blk = h0_ref.shape[0]
    x_all = emb_ref[...].reshape(L * blk, D)
    gi_all = jnp.dot(x_all, wih_ref[...],
                     preferred_element_type=jnp.float32)
    gi_ref[...] = (gi_all + bih_ref[...]).reshape(L, blk, 3 * H)
    whh = whh_ref[...]
    bhh = bhh_ref[...]

    def step(t, h):
        gi = gi_ref[t]
        gh = jnp.dot(h, whh, preferred_element_type=jnp.float32) + bhh
        r = jax.nn.sigmoid(gi[:, :H] + gh[:, :H])
        z = jax.nn.sigmoid(gi[:, H:2 * H] + gh[:, H:2 * H])
        n = jnp.tanh(gi[:, 2 * H:] + r * gh[:, 2 * H:])
        return n + z * (h - n)

    out_ref[...] = jax.lax.fori_loop(0, L, step, h0_ref[...])


def _gru(emb_seq, h0p, wih_t, whh_t, bih, bhh):
    n_pad = emb_seq.shape[1]
    return pl.pallas_call(
        _gru_body,
        grid=(n_pad // BLK,),
        in_specs=[
            pl.BlockSpec((L, BLK, D), lambda i: (0, i, 0)),
            pl.BlockSpec((BLK, H), lambda i: (i, 0)),
            pl.BlockSpec((D, 3 * H), lambda i: (0, 0)),
            pl.BlockSpec((H, 3 * H), lambda i: (0, 0)),
            pl.BlockSpec((1, 3 * H), lambda i: (0, 0)),
            pl.BlockSpec((1, 3 * H), lambda i: (0, 0)),
        ],
        out_specs=pl.BlockSpec((BLK, H), lambda i: (i, 0)),
        out_shape=jax.ShapeDtypeStruct((n_pad, H), jnp.float32),
        scratch_shapes=[pltpu.VMEM((L, BLK, 3 * H), jnp.float32)],
        compiler_params=pltpu.CompilerParams(
            dimension_semantics=("arbitrary",)),
    )(emb_seq, h0p, wih_t, whh_t, bih, bhh)


BLK2 = 1024


def _stage_d_body(x1_ref, w1_ref, dinv_ref, xw1_ref, y1_ref):
    xw1 = jnp.dot(x1_ref[...], w1_ref[...], preferred_element_type=jnp.float32)
    xw1_ref[...] = xw1
    y1_ref[...] = xw1 * dinv_ref[...]


def _stage_d(x1p, W1, dinv_b):
    return pl.pallas_call(
        _stage_d_body,
        grid=(N_ACC // BLK2,),
        in_specs=[
            pl.BlockSpec((BLK2, H), lambda i: (i, 0)),
            pl.BlockSpec((H, H), lambda i: (0, 0)),
            pl.BlockSpec((BLK2, H), lambda i: (i, 0)),
        ],
        out_specs=[
            pl.BlockSpec((BLK2, H), lambda i: (i, 0)),
            pl.BlockSpec((BLK2, H), lambda i: (i, 0)),
        ],
        out_shape=[
            jax.ShapeDtypeStruct((N_ACC, H), jnp.float32),
            jax.ShapeDtypeStruct((N_ACC, H), jnp.float32),
        ],
        compiler_params=pltpu.CompilerParams(
            dimension_semantics=("arbitrary",)),
    )(x1p, W1, dinv_b)


def _stage_f_body(s1a_ref, s1b_ref, xw1_ref, dinv_ref, oh_ref, x1r_ref,
                  w2_ref, b1_ref, xw2_ref, y2_ref, x2r_ref):
    i = pl.program_id(0)
    dinv = dinv_ref[...]
    x2 = dinv * (s1a_ref[0] + s1b_ref[0]) + dinv * dinv * xw1_ref[...] \
        + b1_ref[...]
    x1root = jnp.dot(oh_ref[...], x1r_ref[...],
                     preferred_element_type=jnp.float32)
    xcat = jax.nn.relu(jnp.concatenate([x2, x1root], axis=1))
    xw2 = jnp.dot(xcat, w2_ref[...], preferred_element_type=jnp.float32)
    xw2_ref[...] = xw2
    y2_ref[...] = xw2 * dinv

    @pl.when(i == 0)
    def _():
        x2r_ref[...] = x2[:128, :]


def _stage_f(s1p, xw1, dinv_b, onehot, x1r, W2, b1):
    return pl.pallas_call(
        _stage_f_body,
        grid=(N_ACC // BLK2,),
        in_specs=[
            pl.BlockSpec((1, BLK2, H), lambda i: (0, i, 0)),
            pl.BlockSpec((1, BLK2, H), lambda i: (0, i, 0)),
            pl.BlockSpec((BLK2, H), lambda i: (i, 0)),
            pl.BlockSpec((BLK2, H), lambda i: (i, 0)),
            pl.BlockSpec((BLK2, 128), lambda i: (i, 0)),
            pl.BlockSpec((128, H), lambda i: (0, 0)),
            pl.BlockSpec((2 * H, H), lambda i: (0, 0)),
            pl.BlockSpec((1, H), lambda i: (0, 0)),
        ],
        out_specs=[
            pl.BlockSpec((BLK2, H), lambda i: (i, 0)),
            pl.BlockSpec((BLK2, H), lambda i: (i, 0)),
            pl.BlockSpec((128, H), lambda i: (0, 0)),
        ],
        out_shape=[
            jax.ShapeDtypeStruct((N_ACC, H), jnp.float32),
            jax.ShapeDtypeStruct((N_ACC, H), jnp.float32),
            jax.ShapeDtypeStruct((128, H), jnp.float32),
        ],
        compiler_params=pltpu.CompilerParams(
            dimension_semantics=("arbitrary",)),
    )(s1p.reshape(NC, N_ACC, H)[0:1], s1p.reshape(NC, N_ACC, H)[1:2],
      xw1, dinv_b, onehot, x1r, W2, b1)


def _stage_g_body(s2a_ref, s2b_ref, xw2_ref, dinv_ref, oh_ref, x2r_ref,
                  wf_ref, b2_ref, bf_ref, cntinv_ref, out_ref, acc_ref):
    i = pl.program_id(0)
    dinv = dinv_ref[...]
    x3 = jax.nn.relu(dinv * (s2a_ref[0] + s2b_ref[0])
                     + dinv * dinv * xw2_ref[...] + b2_ref[...])
    x2root = jnp.dot(oh_ref[...], x2r_ref[...],
                     preferred_element_type=jnp.float32)
    xf = jnp.concatenate([x3, x2root], axis=1)
    part = jax.lax.dot_general(oh_ref[...], xf, (((0,), (0,)), ((), ())),
                               preferred_element_type=jnp.float32)

    @pl.when(i == 0)
    def _():
        acc_ref[...] = jnp.zeros_like(acc_ref)

    acc_ref[...] += part

    @pl.when(i == pl.num_programs(0) - 1)
    def _():
        mean = acc_ref[...] * cntinv_ref[...]
        out_ref[...] = jnp.dot(mean, wf_ref[...],
                               preferred_element_type=jnp.float32) \
            + bf_ref[...]


def _stage_g(s2p, xw2, dinv_b, onehot, x2r, Wf, b2, bf, cntinv_b):
    return pl.pallas_call(
        _stage_g_body,
        grid=(N_ACC // BLK2,),
        in_specs=[
            pl.BlockSpec((1, BLK2, H), lambda i: (0, i, 0)),
            pl.BlockSpec((1, BLK2, H), lambda i: (0, i, 0)),
            pl.BlockSpec((BLK2, H), lambda i: (i, 0)),
            pl.BlockSpec((BLK2, H), lambda i: (i, 0)),
            pl.BlockSpec((BLK2, 128), lambda i: (i, 0)),
            pl.BlockSpec((128, H), lambda i: (0, 0)),
            pl.BlockSpec((2 * H, 128), lambda i: (0, 0)),
            pl.BlockSpec((1, H), lambda i: (0, 0)),
            pl.BlockSpec((1, 128), lambda i: (0, 0)),
            pl.BlockSpec((128, 2 * H), lambda i: (0, 0)),
        ],
        out_specs=pl.BlockSpec((128, 128), lambda i: (0, 0)),
        out_shape=jax.ShapeDtypeStruct((128, 128), jnp.float32),
        scratch_shapes=[pltpu.VMEM((128, 2 * H), jnp.float32)],
        compiler_params=pltpu.CompilerParams(
            dimension_semantics=("arbitrary",)),
    )(s2p.reshape(NC, N_ACC, H)[0:1], s2p.reshape(NC, N_ACC, H)[1:2],
      xw2, dinv_b, onehot, x2r, Wf, b2, bf, cntinv_b)


def kernel(user_text, user_feats, graph_node_features, graph_edge_index,
           merged_tree_feature, merged_tree_edge_index, indices,
           emb_table, h0, W_ih, W_hh, b_ih, b_hh,
           W1, b1, W2, b2, Wf, bf):
    n = merged_tree_feature.shape[0]
    b_trees = user_text.shape[0]
    n_pad = ((n + BLK - 1) // BLK) * BLK
    pad = n_pad - n

    emb = jnp.take(emb_table, merged_tree_feature.reshape(-1), axis=0)
    emb_seq = jnp.transpose(emb.reshape(n, L, D), (1, 0, 2))
    emb_seq = jnp.pad(emb_seq, ((0, 0), (0, pad), (0, 0)))
    h0p = jnp.pad(h0, ((0, pad), (0, 0)))
    x1p = _gru(emb_seq, h0p, W_ih.T, W_hh.T, b_ih[None, :], b_hh[None, :])

    src = merged_tree_edge_index[0].astype(jnp.int32)
    dst = merged_tree_edge_index[1].astype(jnp.int32)
    e = src.shape[0]
    quant = NW * CHUNK
    e_pad = ((e + quant - 1) // quant) * quant
    src_p = jnp.concatenate(
        [src, jnp.zeros((e_pad - e,), jnp.int32)])
    dst_p = jnp.concatenate(
        [dst, jnp.full((e_pad - e,), N_ACC - 1, jnp.int32)])

    degp = _deg_sc(dst_p)
    deg = degp[0] + degp[1] + 1.0
    dinv_b = jnp.broadcast_to(jax.lax.rsqrt(deg)[:, None], (N_ACC, H))

    idx_p = jnp.concatenate(
        [indices.astype(jnp.int32),
         jnp.full((N_ACC - n,), b_trees, jnp.int32)])
    onehot = (idx_p[:, None] ==
              jnp.arange(b_trees, dtype=jnp.int32)[None, :]).astype(
                  jnp.float32)
    cnt = jnp.sum(onehot, axis=0)
    cntinv_b = jnp.broadcast_to(
        (1.0 / jnp.clip(cnt, 1.0, None))[:, None], (b_trees, 2 * H))

    xw1, y1 = _stage_d(x1p, W1, dinv_b)
    s1p = _segsum_sc(y1, src_p, dst_p)
    xw2, y2, x2r = _stage_f(s1p, xw1, dinv_b, onehot, x1p[:128], W2,
                            b1[None, :])
    s2p = _segsum_sc(y2, src_p, dst_p)
    wf_p = jnp.pad(Wf, ((0, 0), (0, 128 - Wf.shape[1])))
    bf_p = jnp.pad(bf, ((0, 128 - bf.shape[0]),))
    out = _stage_g(s2p, xw2, dinv_b, onehot, x2r, wf_p, b2[None, :],
                   bf_p[None, :], cntinv_b)
    return out[:, :Wf.shape[1]]
